# baseline (speedup 1.0000x reference)
import functools

import jax
import jax.numpy as jnp
from jax import lax
from jax.experimental import pallas as pl
from jax.experimental.pallas import tpu as pltpu
from jax.experimental.pallas import tpu_sc as plsc

_V, _G, _F = 100000, 2048, 128
_NC, _NS = 2, 16
_NW = _NC * _NS
_HALF = _V // 2
_BLK = 80
_NB_LO = 19
_N20 = 17
_ROWS_LO = _BLK * _NB_LO
_ROWS_BUF = _ROWS_LO + _BLK
_YBLK = 10000
_YGRID = _HALF // _YBLK


def _make_prep(off_blk, with_t):
    def body(nf_ref, gf_ref, w1a_ref, w1b_ref, b1_ref, y_ref, t_ref):
        yrow = lax.dot_general(w1b_ref[...], nf_ref[...],
                               (((1,), (1,)), ((), ())),
                               preferred_element_type=jnp.float32)
        y_ref[...] = yrow.reshape(1, 1, _YBLK)

        if with_t:
            @pl.when(pl.program_id(0) == 0)
            def _():
                gr = jnp.maximum(gf_ref[...], 0.0)
                trow = lax.dot_general(w1a_ref[...], gr,
                                       (((1,), (1,)), ((), ())),
                                       preferred_element_type=jnp.float32)
                t_ref[...] = trow.reshape(1, 1, _G) + b1_ref[0, 0]

    out_specs = [pl.BlockSpec((1, 1, _YBLK), lambda i: (i, 0, 0))]
    out_shape = [jax.ShapeDtypeStruct((_YGRID, 1, _YBLK), jnp.float32)]
    if with_t:
        out_specs.append(pl.BlockSpec((1, 1, _G), lambda i: (0, 0, 0)))
        out_shape.append(jax.ShapeDtypeStruct((1, 1, _G), jnp.float32))
    else:
        body_no_t = body

        def body(nf_ref, gf_ref, w1a_ref, w1b_ref, b1_ref, y_ref):
            body_no_t(nf_ref, gf_ref, w1a_ref, w1b_ref, b1_ref, y_ref, None)

    return pl.pallas_call(
        body,
        grid=(_YGRID,),
        in_specs=[
            pl.BlockSpec((_YBLK, _F), lambda i: (i + off_blk, 0)),
            pl.BlockSpec((_G, _F), lambda i: (0, 0)),
            pl.BlockSpec((1, _F), lambda i: (0, 0)),
            pl.BlockSpec((1, _F), lambda i: (0, 0)),
            pl.BlockSpec((1, 1), lambda i: (0, 0), memory_space=pltpu.SMEM),
        ],
        out_specs=out_specs,
        out_shape=out_shape,
    )


_prep_a = _make_prep(0, True)
_prep_b = _make_prep(_YGRID, False)


def _make_sc_pool(v_off):
    @functools.partial(
        pl.kernel,
        out_type=[
            jax.ShapeDtypeStruct((_NC, _G, _F), jnp.float32),
            jax.ShapeDtypeStruct((_NC, _G), jnp.float32),
        ],
        mesh=plsc.VectorSubcoreMesh(core_axis_name="c", subcore_axis_name="s"),
        compiler_params=pltpu.CompilerParams(needs_layout_passes=False),
        scratch_types=[
            pltpu.VMEM((_G,), jnp.float32),
            pltpu.VMEM((_ROWS_BUF,), jnp.int32),
            pltpu.VMEM((_ROWS_BUF,), jnp.float32),
            pltpu.VMEM((_BLK, _F), jnp.float32),
            pltpu.VMEM((_BLK, _F), jnp.float32),
            pltpu.VMEM((_BLK, _F), jnp.float32),
            pltpu.VMEM((_BLK, _F), jnp.float32),
            pltpu.VMEM((_BLK,), jnp.float32),
            pltpu.VMEM((_BLK,), jnp.float32),
            pltpu.VMEM((_BLK,), jnp.int32),
            pltpu.VMEM((_BLK,), jnp.int32),
            pltpu.VMEM_SHARED((_G, _F), jnp.float32),
            pltpu.VMEM_SHARED((_G,), jnp.float32),
            pltpu.SemaphoreType.DMA,
            pltpu.SemaphoreType.DMA,
            pltpu.SemaphoreType.DMA,
            pltpu.SemaphoreType.DMA,
            pltpu.SemaphoreType.DMA,
            pltpu.SemaphoreType.DMA,
        ],
    )
    def sc_pool(nf_hbm, y_hbm, t_hbm, seg_hbm, znum_hbm, zden_hbm,
                num_out, den_out,
                t_v, seg_v, y_v, nf0, nf1, ob0, ob1, ez0, ez1, ix0, ix1,
                num_sh, den_sh, ld0, ld1, sn0, sn1, sd0, sd1):
        c = lax.axis_index("c")
        s = lax.axis_index("s")
        wid = c * _NS + s
        base_rel = wid * _ROWS_LO + jnp.minimum(wid, _N20) * _BLK
        base = v_off + base_rel
        nblk = jnp.where(wid < _N20, _NB_LO + 1, _NB_LO)
        nf = (nf0, nf1)
        ob = (ob0, ob1)
        ez = (ez0, ez1)
        ix = (ix0, ix1)
        lds = (ld0, ld1)
        sns = (sn0, sn1)
        sds = (sd0, sd1)

        @pl.when(s == 0)
        def _():
            pltpu.sync_copy(znum_hbm, num_sh)
            pltpu.sync_copy(zden_hbm, den_sh)

        pltpu.sync_copy(t_hbm, t_v)
        pltpu.sync_copy(seg_hbm.at[pl.ds(base, _ROWS_LO)],
                        seg_v.at[pl.ds(0, _ROWS_LO)])
        pltpu.sync_copy(y_hbm.at[pl.ds(base_rel, _ROWS_LO)],
                        y_v.at[pl.ds(0, _ROWS_LO)])

        @pl.when(wid < _N20)
        def _():
            pltpu.sync_copy(seg_hbm.at[pl.ds(base + _ROWS_LO, _BLK)],
                            seg_v.at[pl.ds(_ROWS_LO, _BLK)])
            pltpu.sync_copy(y_hbm.at[pl.ds(base_rel + _ROWS_LO, _BLK)],
                            y_v.at[pl.ds(_ROWS_LO, _BLK)])

        plsc.subcore_barrier()

        def start_load(sub, b):
            pltpu.async_copy(nf_hbm.at[pl.ds(base + b * _BLK, _BLK)],
                             nf[sub], lds[sub])

        def wait_load(sub):
            pltpu.make_async_copy(nf_hbm.at[pl.ds(base, _BLK)],
                                  nf[sub], lds[sub]).wait()

        def wait_scats(sub):
            pltpu.make_async_copy(ob[sub], num_sh.at[ix[sub]],
                                  sns[sub]).wait()
            pltpu.make_async_copy(ez[sub], den_sh.at[ix[sub]],
                                  sds[sub]).wait()

        def compute_and_scat(sub, b):
            row0 = b * _BLK
            for g in range(_BLK // 16):
                off = row0 + g * 16
                segv = seg_v[pl.ds(off, 16)]
                tg = plsc.load_gather(t_v, [segv])
                x = y_v[pl.ds(off, 16)] + tg
                ezv = 0.5 + 0.5 * jnp.exp(x)
                ix[sub][pl.ds(g * 16, 16)] = segv
                ez[sub][pl.ds(g * 16, 16)] = ezv
                for j in range(16):
                    w = ezv[j]
                    row = g * 16 + j
                    for k in range(_F // 16):
                        sl = pl.ds(k * 16, 16)
                        ob[sub][row, sl] = nf[sub][row, sl] * w
            pltpu.async_copy(ob[sub], num_sh.at[ix[sub]], sns[sub], add=True)
            pltpu.async_copy(ez[sub], den_sh.at[ix[sub]], sds[sub], add=True)

        start_load(0, 0)
        start_load(1, 1)

        def pair_body(g2, carry):
            for sub in range(2):
                b = 2 * g2 + sub

                @pl.when(g2 > 0)
                def _():
                    wait_scats(sub)

                @pl.when(b < nblk)
                def _():
                    wait_load(sub)
                    compute_and_scat(sub, b)

                @pl.when(b + 2 < nblk)
                def _():
                    start_load(sub, b + 2)
            return carry

        lax.fori_loop(0, (_NB_LO + 2) // 2, pair_body, 0)
        wait_scats(0)

        @pl.when(nblk == _NB_LO + 1)
        def _():
            wait_scats(1)

        plsc.subcore_barrier()

        @pl.when(s == 0)
        def _():
            pltpu.sync_copy(num_sh, num_out.at[c])
            pltpu.sync_copy(den_sh, den_out.at[c])

    return sc_pool


_sc_pool_a = _make_sc_pool(0)
_sc_pool_b = _make_sc_pool(_HALF)


def _final_body(num_a_ref, num_b_ref, den_a_ref, den_b_ref, gf_ref, w2_ref,
                b2_ref, wih_ref, whh_ref, bih_ref, bhh_ref, out_ref):
    num = (num_a_ref[0] + num_a_ref[1]) + (num_b_ref[0] + num_b_ref[1])
    den = (den_a_ref[0] + den_a_ref[1]) + (den_b_ref[0] + den_b_ref[1])
    pos = den > 0.0
    inv = jnp.where(pos, 1.0 / jnp.where(pos, den, 1.0), 0.0)
    wavg = num * inv
    g_repr = lax.dot_general(
        wavg, w2_ref[...], (((1,), (1,)), ((), ())),
        preferred_element_type=jnp.float32)
    g_repr = g_repr + jnp.where(pos, 1.0, 0.0) * b2_ref[...]
    ctx = jnp.where(g_repr > 0.0, g_repr,
                    jnp.exp(jnp.minimum(g_repr, 0.0)) - 1.0)
    gf = gf_ref[...]
    gi = lax.dot_general(ctx, wih_ref[...], (((1,), (1,)), ((), ())),
                         preferred_element_type=jnp.float32) + bih_ref[...]
    gh = lax.dot_general(gf, whh_ref[...], (((1,), (1,)), ((), ())),
                         preferred_element_type=jnp.float32) + bhh_ref[...]
    i_r = gi[:, :_F]
    i_z = gi[:, _F:2 * _F]
    i_n = gi[:, 2 * _F:]
    h_r = gh[:, :_F]
    h_z = gh[:, _F:2 * _F]
    h_n = gh[:, 2 * _F:]
    r = 1.0 / (1.0 + jnp.exp(-(i_r + h_r)))
    u = 1.0 / (1.0 + jnp.exp(-(i_z + h_z)))
    n = jnp.tanh(i_n + r * h_n)
    out_ref[...] = (1.0 - u) * n + u * gf


_final_call = pl.pallas_call(
    _final_body,
    out_shape=jax.ShapeDtypeStruct((_G, _F), jnp.float32),
)


def kernel(node_feats, g_feats, segment_ids, W1, b1, W2, b2,
           W_ih, W_hh, b_ih, b_hh):
    seg = segment_ids.astype(jnp.int32)
    w1a = W1[:, :_F]
    w1b = W1[:, _F:]
    b1_2d = b1.reshape(1, 1)
    y_a, t = _prep_a(node_feats, g_feats, w1a, w1b, b1_2d)
    (y_b,) = _prep_b(node_feats, g_feats, w1a, w1b, b1_2d)
    znum = jnp.zeros((_G, _F), jnp.float32)
    zden = jnp.zeros((_G,), jnp.float32)
    t1 = t.reshape(_G)
    num_a, den_a = _sc_pool_a(node_feats, y_a.reshape(_HALF), t1, seg,
                              znum, zden)
    num_b, den_b = _sc_pool_b(node_feats, y_b.reshape(_HALF), t1, seg,
                              znum, zden)
    out = _final_call(num_a, num_b, den_a.reshape(_NC, _G, 1),
                      den_b.reshape(_NC, _G, 1), g_feats, W2,
                      b2.reshape(1, _F), W_ih, W_hh,
                      b_ih.reshape(1, 3 * _F), b_hh.reshape(1, 3 * _F))
    return out
